# baseline (speedup 1.0000x reference)
import functools

import jax
import jax.numpy as jnp
import numpy as np
from jax import lax
from jax.experimental import pallas as pl
from jax.experimental.pallas import tpu as pltpu
from jax.experimental.pallas import tpu_sc as plsc

B = 1; Q = 8; KV = 4096; H = 32; HD = 128; D = 4096; FF = 11008
L = KV + Q
LP = 4224
NBLK = LP // HD
NUM_REMAIN = L - int(L * 0.9)
ROPE_BASE = 10000.0
INV_SQRT_HD = 1.0 / np.sqrt(HD).astype(np.float32)
NEG = float(jnp.finfo(jnp.float32).min)
F32 = jnp.float32

_DB = 256
_FB = 256


def _rot_half(x):
    rolled = jnp.roll(x, HD // 2, axis=-1)
    lane = jax.lax.broadcasted_iota(jnp.int32, x.shape, len(x.shape) - 1)
    return jnp.where(lane < HD // 2, -rolled, rolled)


def _mm(a, b, ca, cb):
    return jax.lax.dot_general(a, b, (((ca,), (cb,)), ((), ())),
                               preferred_element_type=F32)


def _rms(x, w):
    ms = jnp.mean(x * x, axis=-1, keepdims=True)
    return x * jax.lax.rsqrt(ms + 1e-6) * w


def _qkv_body(h_ref, w1_ref, wq_ref, wk_ref, wv_ref, q_ref, k_ref, v_ref):
    hn = _rms(h_ref[...], w1_ref[...])
    q_ref[...] = _mm(hn, wq_ref[...], 1, 1)
    k_ref[...] = _mm(hn, wk_ref[...], 1, 1)
    v_ref[...] = _mm(hn, wv_ref[...], 1, 1)


def _qkv_call(hid, w1, Wq, Wk, Wv):
    n = D // _DB
    return pl.pallas_call(
        _qkv_body,
        grid=(n,),
        in_specs=[
            pl.BlockSpec((Q, D), lambda i: (0, 0)),
            pl.BlockSpec((1, D), lambda i: (0, 0)),
            pl.BlockSpec((_DB, D), lambda i: (i, 0)),
            pl.BlockSpec((_DB, D), lambda i: (i, 0)),
            pl.BlockSpec((_DB, D), lambda i: (i, 0)),
        ],
        out_specs=[pl.BlockSpec((Q, _DB), lambda i: (0, i))] * 3,
        out_shape=[jax.ShapeDtypeStruct((Q, D), F32)] * 3,
    )(hid, w1, Wq, Wk, Wv)


def _hp_body(q_ref, k_ref, r1_ref, r2_ref, cos_ref, sin_ref,
             qr_ref, qh_ref, dn_ref, rn_ref):
    q = q_ref[...].reshape(Q, HD)
    k = k_ref[...].reshape(Q, HD)
    cos = cos_ref[...]
    sin = sin_ref[...]
    r1 = r1_ref[...].reshape(HD, HD)
    r2 = r2_ref[...].reshape(HD, HD)
    qr = q * cos + _rot_half(q) * sin
    kr = k * cos + _rot_half(k) * sin
    qi = _mm(jax.nn.silu(_mm(qr, r1, 1, 0)), r2, 1, 0)
    ki = _mm(jax.nn.silu(_mm(kr, r1, 1, 0)), r2, 1, 0)
    qs = jnp.sign(qi)
    ks = jnp.sign(ki)
    qr_ref[...] = qr.reshape(1, Q, HD)
    qh_ref[...] = qs.reshape(1, Q, HD)
    dn_ref[...] = _mm(qs, ks, 1, 1).reshape(1, Q, Q)
    rn_ref[...] = (_mm(qr, kr, 1, 1) * INV_SQRT_HD).reshape(1, Q, Q)


def _hp_call(qh, kh, r1, r2, cos_n, sin_n):
    return pl.pallas_call(
        _hp_body,
        grid=(H,),
        in_specs=[
            pl.BlockSpec((1, Q, HD), lambda i: (i, 0, 0)),
            pl.BlockSpec((1, Q, HD), lambda i: (i, 0, 0)),
            pl.BlockSpec((1, HD, HD), lambda i: (i, 0, 0)),
            pl.BlockSpec((1, HD, HD), lambda i: (i, 0, 0)),
            pl.BlockSpec((Q, HD), lambda i: (0, 0)),
            pl.BlockSpec((Q, HD), lambda i: (0, 0)),
        ],
        out_specs=[
            pl.BlockSpec((1, Q, HD), lambda i: (i, 0, 0)),
            pl.BlockSpec((1, Q, HD), lambda i: (i, 0, 0)),
            pl.BlockSpec((1, Q, Q), lambda i: (i, 0, 0)),
            pl.BlockSpec((1, Q, Q), lambda i: (i, 0, 0)),
        ],
        out_shape=[
            jax.ShapeDtypeStruct((H, Q, HD), F32),
            jax.ShapeDtypeStruct((H, Q, HD), F32),
            jax.ShapeDtypeStruct((H, Q, Q), F32),
            jax.ShapeDtypeStruct((H, Q, Q), F32),
        ],
    )(qh, kh, r1, r2, cos_n, sin_n)


def _score_body(kc_ref, r1_ref, r2_ref, cos_ref, sin_ref, qr_ref, qh_ref,
                d_ref, r_ref):
    k = kc_ref[...].reshape(KV, HD)
    kr = k * cos_ref[...] + _rot_half(k) * sin_ref[...]
    r1 = r1_ref[...].reshape(HD, HD)
    r2 = r2_ref[...].reshape(HD, HD)
    ki = _mm(jax.nn.silu(_mm(kr, r1, 1, 0)), r2, 1, 0)
    ks = jnp.sign(ki)
    qh = qh_ref[...].reshape(Q, HD)
    qr = qr_ref[...].reshape(Q, HD)
    d_ref[...] = _mm(qh, ks, 1, 1).reshape(1, Q, KV)
    r_ref[...] = (_mm(qr, kr, 1, 1) * INV_SQRT_HD).reshape(1, Q, KV)


def _score_call(kc, r1, r2, cos_c, sin_c, q_rope, q_hash, off, nh):
    return pl.pallas_call(
        _score_body,
        grid=(nh,),
        in_specs=[
            pl.BlockSpec((1, KV, HD), lambda i: (i + off, 0, 0)),
            pl.BlockSpec((1, HD, HD), lambda i: (i + off, 0, 0)),
            pl.BlockSpec((1, HD, HD), lambda i: (i + off, 0, 0)),
            pl.BlockSpec((KV, HD), lambda i: (0, 0)),
            pl.BlockSpec((KV, HD), lambda i: (0, 0)),
            pl.BlockSpec((1, Q, HD), lambda i: (i + off, 0, 0)),
            pl.BlockSpec((1, Q, HD), lambda i: (i + off, 0, 0)),
        ],
        out_specs=[
            pl.BlockSpec((1, Q, KV), lambda i: (i, 0, 0)),
            pl.BlockSpec((1, Q, KV), lambda i: (i, 0, 0)),
        ],
        out_shape=[
            jax.ShapeDtypeStruct((nh, Q, KV), F32),
            jax.ShapeDtypeStruct((nh, Q, KV), F32),
        ],
    )(kc, r1, r2, cos_c, sin_c, q_rope, q_hash)


_ROWS = H * Q
_NW = 32
_RPW = _ROWS // _NW
_NBIN = 272
_NCH = KV // 16


def _sc_sel_body(rpw, dc_hbm, dn_hbm, mc_hbm, mn_hbm, row_v, tail_v, bins_v):
    wid = lax.axis_index("s") * 2 + lax.axis_index("c")
    lane = lax.iota(jnp.int32, 16)
    lanef = lane.astype(F32)
    lane_off = lane * _NBIN
    ones = jnp.full((16,), 1.0, F32)
    zeros = jnp.zeros((16,), F32)
    krem = float(NUM_REMAIN)
    nchunk = _NBIN // 16

    def _merged(i):
        m = bins_v[pl.ds(i * 16, 16)]
        for l in range(1, 16):
            m = m + bins_v[pl.ds(l * _NBIN + i * 16, 16)]
        return m

    def _row(r, row_carry):
        rg = wid * rpw + r
        tail_v[...] = jnp.full((16,), -1000.0, F32)
        pltpu.sync_copy(dc_hbm.at[pl.ds(rg * KV, KV)], row_v)
        pltpu.sync_copy(dn_hbm.at[pl.ds(rg * Q, Q)], tail_v.at[pl.ds(0, Q)])

        def _zero(i, c):
            bins_v[pl.ds(i * 16, 16)] = zeros
            return c

        lax.fori_loop(0, _NBIN, _zero, 0, unroll=4)

        def _hist(c, carry):
            v = row_v[pl.ds(c * 16, 16)]
            idx = v.astype(jnp.int32) + 128 + lane_off
            plsc.addupdate_scatter(bins_v, [idx], ones)
            return carry

        lax.fori_loop(0, _NCH, _hist, 0, unroll=8)
        tv = tail_v[...]
        tidx = tv.astype(jnp.int32) + 128 + lane_off
        plsc.addupdate_scatter(bins_v, [tidx], ones, mask=tv > -999.0)

        def _scan(j, st):
            t_run, cge_run, carry = st
            i = nchunk - 1 - j
            m = _merged(i)
            cs = plsc.cumsum(lax.rev(m, (0,))) + carry
            bv = (i * 16 + 15 - 128).astype(F32) - lanef
            ok = cs >= krem
            t_run = jnp.maximum(t_run, jnp.where(ok, bv, -999.0))
            cge_run = jnp.minimum(cge_run, jnp.where(ok, cs, 1e9))
            return t_run, cge_run, carry + jnp.sum(m)

        t_run, cge_run, _ = lax.fori_loop(
            0, nchunk, _scan,
            (jnp.full((16,), -999.0, F32), jnp.full((16,), 1e9, F32),
             jnp.float32(0.0)))
        thr = jnp.max(t_run)
        cge = jnp.min(cge_run)
        bidx = thr.astype(jnp.int32) + 128
        cbase = (bidx // 16) * 16
        blane = bidx - cbase

        def _ecnt(l, acc):
            m = bins_v[pl.ds(l * _NBIN + cbase, 16)]
            return acc + jnp.sum(jnp.where(lane == blane, m, 0.0))

        ecnt = lax.fori_loop(0, 16, _ecnt, jnp.float32(0.0))
        rrem = krem - (cge - ecnt)

        def _sel(c, base):
            v = row_v[pl.ds(c * 16, 16)]
            eq = v == thr

            def _tie(_):
                eqf = jnp.where(eq, 1.0, 0.0)
                csum = plsc.cumsum(eqf) + base
                keep = (v > thr) | (eq & (csum <= rrem))
                return jnp.where(keep, 0.0, NEG), base + jnp.sum(eqf)

            def _fast(_):
                return jnp.where(v > thr, 0.0, NEG), base

            maskv, nbase = lax.cond(jnp.any(eq), _tie, _fast, 0)
            row_v[pl.ds(c * 16, 16)] = maskv
            return nbase

        base = lax.fori_loop(0, _NCH, _sel, jnp.float32(0.0), unroll=4)
        tv = tail_v[...]
        eq = jnp.where(tv == thr, 1.0, 0.0)
        csum = plsc.cumsum(eq) + base
        keep = (tv > thr) | ((eq > 0.5) & (csum <= rrem))
        tail_v[...] = jnp.where(keep, 0.0, NEG)

        pltpu.sync_copy(row_v, mc_hbm.at[pl.ds(rg * KV, KV)])
        pltpu.sync_copy(tail_v.at[pl.ds(0, Q)], mn_hbm.at[pl.ds(rg * Q, Q)])
        return row_carry

    lax.fori_loop(0, rpw, _row, 0)


def _sc_sel_call(draft_c, draft_new):
    nh = draft_c.shape[0]
    rows = nh * Q
    rpw = rows // _NW
    fn = functools.partial(
        pl.kernel,
        out_type=[
            jax.ShapeDtypeStruct((rows * KV,), F32),
            jax.ShapeDtypeStruct((rows * Q,), F32),
        ],
        mesh=plsc.VectorSubcoreMesh(core_axis_name="c", subcore_axis_name="s"),
        compiler_params=pltpu.CompilerParams(needs_layout_passes=False),
        scratch_types=[
            pltpu.VMEM((KV,), F32),
            pltpu.VMEM((16,), F32),
            pltpu.VMEM((16 * _NBIN,), F32),
        ],
    )(functools.partial(_sc_sel_body, rpw))
    mc, mn = fn(draft_c.reshape(rows * KV), draft_new.reshape(rows * Q))
    return mc.reshape(nh, Q, KV), mn.reshape(nh, Q, Q)


_AH = 4
_AR = _AH * Q


def _att_body(mc_ref, mn_ref, rc_ref, rn_ref, vc_ref, vn_ref, o_ref):
    mc = mc_ref[...].reshape(_AR, KV)
    mn = mn_ref[...].reshape(_AR, Q)
    rc = rc_ref[...].reshape(_AR, KV)
    rn = rn_ref[...].reshape(_AR, Q)
    masked = jnp.concatenate(
        [rc + mc, rn + mn, jnp.full((_AR, LP - L), NEG, F32)], axis=-1)
    m = jnp.max(masked, axis=-1, keepdims=True)
    p = jnp.exp(masked - m)
    denom = jnp.sum(p, axis=-1, keepdims=True)
    vc = vc_ref[...]
    vn = vn_ref[...]
    outs = []
    for hh in range(_AH):
        ph = p[hh * Q:(hh + 1) * Q]
        att = _mm(ph[:, :KV], vc[hh], 1, 0) + _mm(ph[:, KV:LP], vn[hh], 1, 0)
        outs.append(att / denom[hh * Q:(hh + 1) * Q])
    o_ref[...] = jnp.concatenate(outs, axis=0).reshape(_AH, Q, HD)


def _att_call(mask_c, mask_new, real_c, real_new, vc, v_new_pad, off, nh):
    return pl.pallas_call(
        _att_body,
        grid=(nh // _AH,),
        in_specs=[
            pl.BlockSpec((_AH, Q, KV), lambda i: (i, 0, 0)),
            pl.BlockSpec((_AH, Q, Q), lambda i: (i, 0, 0)),
            pl.BlockSpec((_AH, Q, KV), lambda i: (i, 0, 0)),
            pl.BlockSpec((_AH, Q, Q), lambda i: (i, 0, 0)),
            pl.BlockSpec((_AH, KV, HD), lambda i: (i + off // _AH, 0, 0)),
            pl.BlockSpec((_AH, HD, HD), lambda i: (i + off // _AH, 0, 0)),
        ],
        out_specs=pl.BlockSpec((_AH, Q, HD), lambda i: (i, 0, 0)),
        out_shape=jax.ShapeDtypeStruct((nh, Q, HD), F32),
    )(mask_c, mask_new, real_c, real_new, vc, v_new_pad)


def _op_body(a_ref, hid_ref, wo_ref, o_ref):
    o_ref[...] = _mm(a_ref[...], wo_ref[...], 1, 1) + hid_ref[...]


def _op_call(attn_f, hid, Wo):
    n = D // _DB
    return pl.pallas_call(
        _op_body,
        grid=(n,),
        in_specs=[
            pl.BlockSpec((Q, D), lambda i: (0, 0)),
            pl.BlockSpec((Q, _DB), lambda i: (0, i)),
            pl.BlockSpec((_DB, D), lambda i: (i, 0)),
        ],
        out_specs=pl.BlockSpec((Q, _DB), lambda i: (0, i)),
        out_shape=jax.ShapeDtypeStruct((Q, D), F32),
    )(attn_f, hid, Wo)


def _mlp_body(h_ref, w2_ref, wg_ref, wu_ref, wd_ref, o_ref):
    i = pl.program_id(0)
    h = h_ref[...]
    hn = _rms(h, w2_ref[...])
    g = jax.nn.silu(_mm(hn, wg_ref[...], 1, 1))
    u = _mm(hn, wu_ref[...], 1, 1)
    part = _mm(g * u, wd_ref[...], 1, 1)

    @pl.when(i == 0)
    def _():
        o_ref[...] = h + part

    @pl.when(i > 0)
    def _():
        o_ref[...] += part


def _mlp_call(h_res, w2, Wg, Wu, Wd):
    n = FF // _FB
    return pl.pallas_call(
        _mlp_body,
        grid=(n,),
        in_specs=[
            pl.BlockSpec((Q, D), lambda i: (0, 0)),
            pl.BlockSpec((1, D), lambda i: (0, 0)),
            pl.BlockSpec((_FB, D), lambda i: (i, 0)),
            pl.BlockSpec((_FB, D), lambda i: (i, 0)),
            pl.BlockSpec((D, _FB), lambda i: (0, i)),
        ],
        out_specs=pl.BlockSpec((Q, D), lambda i: (0, 0)),
        out_shape=jax.ShapeDtypeStruct((Q, D), F32),
    )(h_res, w2, Wg, Wu, Wd)


def kernel(hidden_states, key_cache, value_cache, Wq, Wk, Wv, Wo,
           rot_mat1, rot_mat2, ln1_w, ln2_w, Wg, Wu, Wd):
    hid = hidden_states.reshape(Q, D)
    kc = key_cache.reshape(H, KV, HD)
    vc = value_cache.reshape(H, KV, HD)
    r1 = rot_mat1.reshape(H, HD, HD)
    r2 = rot_mat2.reshape(H, HD, HD)
    w1 = ln1_w.reshape(1, D)
    w2 = ln2_w.reshape(1, D)

    inv_freq = 1.0 / (ROPE_BASE ** (jnp.arange(0, HD, 2, dtype=F32) / HD))
    t = jnp.arange(L, dtype=F32)
    freqs = jnp.outer(t, inv_freq)
    emb = jnp.concatenate([freqs, freqs], axis=-1)
    cos = jnp.cos(emb)
    sin = jnp.sin(emb)
    cos_c, cos_n = cos[:KV], cos[KV:]
    sin_c, sin_n = sin[:KV], sin[KV:]

    q_f, k_f, v_f = _qkv_call(hid, w1, Wq, Wk, Wv)
    qh = q_f.reshape(Q, H, HD).transpose(1, 0, 2)
    kh = k_f.reshape(Q, H, HD).transpose(1, 0, 2)
    vh = v_f.reshape(Q, H, HD).transpose(1, 0, 2)
    v_new_pad = jnp.pad(vh, ((0, 0), (0, HD - Q), (0, 0)))

    q_rope, q_hash, draft_new, real_new = _hp_call(qh, kh, r1, r2, cos_n, sin_n)

    halves = []
    nh = H // 2
    for half in range(2):
        off = half * nh
        d_c, r_c = _score_call(kc, r1, r2, cos_c, sin_c, q_rope, q_hash,
                               off, nh)
        m_c, m_n = _sc_sel_call(d_c, draft_new[off:off + nh])
        halves.append((m_c, m_n, r_c, off))
    attn = jnp.concatenate(
        [_att_call(m_c, m_n, r_c, real_new[off:off + nh], vc, v_new_pad,
                   off, nh)
         for (m_c, m_n, r_c, off) in halves], axis=0)
    attn_f = attn.transpose(1, 0, 2).reshape(Q, D)
    h_res = _op_call(attn_f, hid, Wo)
    out = _mlp_call(h_res, w2, Wg, Wu, Wd)
    return out.reshape(B, Q, D)


# speedup vs baseline: 1.0342x; 1.0342x over previous
import functools

import jax
import jax.numpy as jnp
import numpy as np
from jax import lax
from jax.experimental import pallas as pl
from jax.experimental.pallas import tpu as pltpu
from jax.experimental.pallas import tpu_sc as plsc

B = 1; Q = 8; KV = 4096; H = 32; HD = 128; D = 4096; FF = 11008
L = KV + Q
LP = 4224
NBLK = LP // HD
NUM_REMAIN = L - int(L * 0.9)
ROPE_BASE = 10000.0
INV_SQRT_HD = 1.0 / np.sqrt(HD).astype(np.float32)
NEG = float(jnp.finfo(jnp.float32).min)
F32 = jnp.float32

_DB = 256
_FB = 256


def _rot_half(x):
    rolled = jnp.roll(x, HD // 2, axis=-1)
    lane = jax.lax.broadcasted_iota(jnp.int32, x.shape, len(x.shape) - 1)
    return jnp.where(lane < HD // 2, -rolled, rolled)


def _mm(a, b, ca, cb):
    return jax.lax.dot_general(a, b, (((ca,), (cb,)), ((), ())),
                               preferred_element_type=F32)


def _rms(x, w):
    ms = jnp.mean(x * x, axis=-1, keepdims=True)
    return x * jax.lax.rsqrt(ms + 1e-6) * w


def _qkv_body(h_ref, w1_ref, wq_ref, wk_ref, wv_ref, q_ref, k_ref, v_ref):
    hn = _rms(h_ref[...], w1_ref[...])
    q_ref[...] = _mm(hn, wq_ref[...], 1, 1)
    k_ref[...] = _mm(hn, wk_ref[...], 1, 1)
    v_ref[...] = _mm(hn, wv_ref[...], 1, 1)


def _qkv_call(hid, w1, Wq, Wk, Wv):
    n = D // _DB
    return pl.pallas_call(
        _qkv_body,
        grid=(n,),
        in_specs=[
            pl.BlockSpec((Q, D), lambda i: (0, 0)),
            pl.BlockSpec((1, D), lambda i: (0, 0)),
            pl.BlockSpec((_DB, D), lambda i: (i, 0)),
            pl.BlockSpec((_DB, D), lambda i: (i, 0)),
            pl.BlockSpec((_DB, D), lambda i: (i, 0)),
        ],
        out_specs=[pl.BlockSpec((Q, _DB), lambda i: (0, i))] * 3,
        out_shape=[jax.ShapeDtypeStruct((Q, D), F32)] * 3,
    )(hid, w1, Wq, Wk, Wv)


def _hp_body(q_ref, k_ref, r1_ref, r2_ref, cos_ref, sin_ref,
             qr_ref, qh_ref, dn_ref, rn_ref):
    q = q_ref[...].reshape(Q, HD)
    k = k_ref[...].reshape(Q, HD)
    cos = cos_ref[...]
    sin = sin_ref[...]
    r1 = r1_ref[...].reshape(HD, HD)
    r2 = r2_ref[...].reshape(HD, HD)
    qr = q * cos + _rot_half(q) * sin
    kr = k * cos + _rot_half(k) * sin
    qi = _mm(jax.nn.silu(_mm(qr, r1, 1, 0)), r2, 1, 0)
    ki = _mm(jax.nn.silu(_mm(kr, r1, 1, 0)), r2, 1, 0)
    qs = jnp.sign(qi)
    ks = jnp.sign(ki)
    qr_ref[...] = qr.reshape(1, Q, HD)
    qh_ref[...] = qs.reshape(1, Q, HD)
    dn_ref[...] = _mm(qs, ks, 1, 1).reshape(1, Q, Q)
    rn_ref[...] = (_mm(qr, kr, 1, 1) * INV_SQRT_HD).reshape(1, Q, Q)


def _hp_call(qh, kh, r1, r2, cos_n, sin_n):
    return pl.pallas_call(
        _hp_body,
        grid=(H,),
        in_specs=[
            pl.BlockSpec((1, Q, HD), lambda i: (i, 0, 0)),
            pl.BlockSpec((1, Q, HD), lambda i: (i, 0, 0)),
            pl.BlockSpec((1, HD, HD), lambda i: (i, 0, 0)),
            pl.BlockSpec((1, HD, HD), lambda i: (i, 0, 0)),
            pl.BlockSpec((Q, HD), lambda i: (0, 0)),
            pl.BlockSpec((Q, HD), lambda i: (0, 0)),
        ],
        out_specs=[
            pl.BlockSpec((1, Q, HD), lambda i: (i, 0, 0)),
            pl.BlockSpec((1, Q, HD), lambda i: (i, 0, 0)),
            pl.BlockSpec((1, Q, Q), lambda i: (i, 0, 0)),
            pl.BlockSpec((1, Q, Q), lambda i: (i, 0, 0)),
        ],
        out_shape=[
            jax.ShapeDtypeStruct((H, Q, HD), F32),
            jax.ShapeDtypeStruct((H, Q, HD), F32),
            jax.ShapeDtypeStruct((H, Q, Q), F32),
            jax.ShapeDtypeStruct((H, Q, Q), F32),
        ],
    )(qh, kh, r1, r2, cos_n, sin_n)


def _bd(a, b):
    z = jnp.zeros((HD, HD), F32)
    return jnp.concatenate(
        [jnp.concatenate([a, z], axis=1), jnp.concatenate([z, b], axis=1)],
        axis=0)


def _score_body(kc_ref, r1_ref, r2_ref, cos_ref, sin_ref, qr_ref, qh_ref,
                d_ref, r_ref):
    k2 = kc_ref[...]
    cos = cos_ref[...]
    sin = sin_ref[...]
    kra = k2[0] * cos + _rot_half(k2[0]) * sin
    krb = k2[1] * cos + _rot_half(k2[1]) * sin
    kr2 = jnp.concatenate([kra, krb], axis=1)
    r1 = r1_ref[...]
    r2 = r2_ref[...]
    bd1 = _bd(r1[0], r1[1])
    bd2 = _bd(r2[0], r2[1])
    ki2 = _mm(jax.nn.silu(_mm(kr2, bd1, 1, 0)), bd2, 1, 0)
    ks2 = jnp.sign(ki2)
    qh2 = qh_ref[...]
    qr2 = qr_ref[...]
    qhbd = _bd2q(qh2[0], qh2[1])
    qrbd = _bd2q(qr2[0], qr2[1])
    d_ref[...] = _mm(qhbd, ks2, 1, 1).reshape(2, Q, KV)
    r_ref[...] = (_mm(qrbd, kr2, 1, 1) * INV_SQRT_HD).reshape(2, Q, KV)


def _bd2q(a, b):
    z = jnp.zeros((Q, HD), F32)
    return jnp.concatenate(
        [jnp.concatenate([a, z], axis=1), jnp.concatenate([z, b], axis=1)],
        axis=0)


def _score_call(kc, r1, r2, cos_c, sin_c, q_rope, q_hash, off, nh):
    return pl.pallas_call(
        _score_body,
        grid=(nh // 2,),
        in_specs=[
            pl.BlockSpec((2, KV, HD), lambda i: (i + off // 2, 0, 0)),
            pl.BlockSpec((2, HD, HD), lambda i: (i + off // 2, 0, 0)),
            pl.BlockSpec((2, HD, HD), lambda i: (i + off // 2, 0, 0)),
            pl.BlockSpec((KV, HD), lambda i: (0, 0)),
            pl.BlockSpec((KV, HD), lambda i: (0, 0)),
            pl.BlockSpec((2, Q, HD), lambda i: (i + off // 2, 0, 0)),
            pl.BlockSpec((2, Q, HD), lambda i: (i + off // 2, 0, 0)),
        ],
        out_specs=[
            pl.BlockSpec((2, Q, KV), lambda i: (i, 0, 0)),
            pl.BlockSpec((2, Q, KV), lambda i: (i, 0, 0)),
        ],
        out_shape=[
            jax.ShapeDtypeStruct((nh, Q, KV), F32),
            jax.ShapeDtypeStruct((nh, Q, KV), F32),
        ],
    )(kc, r1, r2, cos_c, sin_c, q_rope, q_hash)


_ROWS = H * Q
_NW = 32
_RPW = _ROWS // _NW
_NBIN = 272
_NCH = KV // 16


def _sc_sel_body(rpw, dc_hbm, dn_hbm, mc_hbm, mn_hbm, row_v, tail_v, bins_v):
    wid = lax.axis_index("s") * 2 + lax.axis_index("c")
    lane = lax.iota(jnp.int32, 16)
    lanef = lane.astype(F32)
    lane_off = lane * _NBIN
    ones = jnp.full((16,), 1.0, F32)
    zeros = jnp.zeros((16,), F32)
    krem = float(NUM_REMAIN)
    nchunk = _NBIN // 16

    def _merged(i):
        m = bins_v[pl.ds(i * 16, 16)]
        for l in range(1, 16):
            m = m + bins_v[pl.ds(l * _NBIN + i * 16, 16)]
        return m

    def _row(r, row_carry):
        rg = wid * rpw + r
        tail_v[...] = jnp.full((16,), -1000.0, F32)
        pltpu.sync_copy(dc_hbm.at[pl.ds(rg * KV, KV)], row_v)
        pltpu.sync_copy(dn_hbm.at[pl.ds(rg * Q, Q)], tail_v.at[pl.ds(0, Q)])

        def _zero(i, c):
            bins_v[pl.ds(i * 16, 16)] = zeros
            return c

        lax.fori_loop(0, _NBIN, _zero, 0, unroll=4)

        def _hist(c, carry):
            v = row_v[pl.ds(c * 16, 16)]
            idx = v.astype(jnp.int32) + 128 + lane_off
            plsc.addupdate_scatter(bins_v, [idx], ones)
            return carry

        lax.fori_loop(0, _NCH, _hist, 0, unroll=8)
        tv = tail_v[...]
        tidx = tv.astype(jnp.int32) + 128 + lane_off
        plsc.addupdate_scatter(bins_v, [tidx], ones, mask=tv > -999.0)

        def _scan(j, st):
            t_run, cge_run, carry = st
            i = nchunk - 1 - j
            m = _merged(i)
            cs = plsc.cumsum(lax.rev(m, (0,))) + carry
            bv = (i * 16 + 15 - 128).astype(F32) - lanef
            ok = cs >= krem
            t_run = jnp.maximum(t_run, jnp.where(ok, bv, -999.0))
            cge_run = jnp.minimum(cge_run, jnp.where(ok, cs, 1e9))
            return t_run, cge_run, carry + jnp.sum(m)

        t_run, cge_run, _ = lax.fori_loop(
            0, nchunk, _scan,
            (jnp.full((16,), -999.0, F32), jnp.full((16,), 1e9, F32),
             jnp.float32(0.0)))
        thr = jnp.max(t_run)
        cge = jnp.min(cge_run)
        bidx = thr.astype(jnp.int32) + 128
        cbase = (bidx // 16) * 16
        blane = bidx - cbase

        def _ecnt(l, acc):
            m = bins_v[pl.ds(l * _NBIN + cbase, 16)]
            return acc + jnp.sum(jnp.where(lane == blane, m, 0.0))

        ecnt = lax.fori_loop(0, 16, _ecnt, jnp.float32(0.0))
        rrem = krem - (cge - ecnt)

        def _sel(c, base):
            v = row_v[pl.ds(c * 16, 16)]
            eq = jnp.where(v == thr, 1.0, 0.0)
            csum = plsc.cumsum(eq) + base
            keep = (v > thr) | ((eq > 0.5) & (csum <= rrem))
            row_v[pl.ds(c * 16, 16)] = jnp.where(keep, 0.0, NEG)
            return base + jnp.sum(eq)

        base = lax.fori_loop(0, _NCH, _sel, jnp.float32(0.0), unroll=8)
        tv = tail_v[...]
        eq = jnp.where(tv == thr, 1.0, 0.0)
        csum = plsc.cumsum(eq) + base
        keep = (tv > thr) | ((eq > 0.5) & (csum <= rrem))
        tail_v[...] = jnp.where(keep, 0.0, NEG)

        pltpu.sync_copy(row_v, mc_hbm.at[pl.ds(rg * KV, KV)])
        pltpu.sync_copy(tail_v.at[pl.ds(0, Q)], mn_hbm.at[pl.ds(rg * Q, Q)])
        return row_carry

    lax.fori_loop(0, rpw, _row, 0)


def _sc_sel_call(draft_c, draft_new):
    nh = draft_c.shape[0]
    rows = nh * Q
    rpw = rows // _NW
    fn = functools.partial(
        pl.kernel,
        out_type=[
            jax.ShapeDtypeStruct((rows * KV,), F32),
            jax.ShapeDtypeStruct((rows * Q,), F32),
        ],
        mesh=plsc.VectorSubcoreMesh(core_axis_name="c", subcore_axis_name="s"),
        compiler_params=pltpu.CompilerParams(needs_layout_passes=False),
        scratch_types=[
            pltpu.VMEM((KV,), F32),
            pltpu.VMEM((16,), F32),
            pltpu.VMEM((16 * _NBIN,), F32),
        ],
    )(functools.partial(_sc_sel_body, rpw))
    mc, mn = fn(draft_c.reshape(rows * KV), draft_new.reshape(rows * Q))
    return mc.reshape(nh, Q, KV), mn.reshape(nh, Q, Q)


_AH = 4
_AR = _AH * Q


def _att_body(mc_ref, mn_ref, rc_ref, rn_ref, vc_ref, vn_ref, o_ref):
    mc = mc_ref[...].reshape(_AR, KV)
    mn = mn_ref[...].reshape(_AR, Q)
    rc = rc_ref[...].reshape(_AR, KV)
    rn = rn_ref[...].reshape(_AR, Q)
    masked = jnp.concatenate(
        [rc + mc, rn + mn, jnp.full((_AR, LP - L), NEG, F32)], axis=-1)
    m = jnp.max(masked, axis=-1, keepdims=True)
    p = jnp.exp(masked - m)
    denom = jnp.sum(p, axis=-1, keepdims=True)
    vc = vc_ref[...]
    vn = vn_ref[...]
    outs = []
    for hh in range(_AH):
        ph = p[hh * Q:(hh + 1) * Q]
        att = _mm(ph[:, :KV], vc[hh], 1, 0) + _mm(ph[:, KV:LP], vn[hh], 1, 0)
        outs.append(att / denom[hh * Q:(hh + 1) * Q])
    o_ref[...] = jnp.concatenate(outs, axis=0).reshape(_AH, Q, HD)


def _att_call(mask_c, mask_new, real_c, real_new, vc, v_new_pad, off, nh):
    return pl.pallas_call(
        _att_body,
        grid=(nh // _AH,),
        in_specs=[
            pl.BlockSpec((_AH, Q, KV), lambda i: (i, 0, 0)),
            pl.BlockSpec((_AH, Q, Q), lambda i: (i, 0, 0)),
            pl.BlockSpec((_AH, Q, KV), lambda i: (i, 0, 0)),
            pl.BlockSpec((_AH, Q, Q), lambda i: (i, 0, 0)),
            pl.BlockSpec((_AH, KV, HD), lambda i: (i + off // _AH, 0, 0)),
            pl.BlockSpec((_AH, HD, HD), lambda i: (i + off // _AH, 0, 0)),
        ],
        out_specs=pl.BlockSpec((_AH, Q, HD), lambda i: (i, 0, 0)),
        out_shape=jax.ShapeDtypeStruct((nh, Q, HD), F32),
    )(mask_c, mask_new, real_c, real_new, vc, v_new_pad)


def _op_body(a_ref, hid_ref, wo_ref, o_ref):
    o_ref[...] = _mm(a_ref[...], wo_ref[...], 1, 1) + hid_ref[...]


def _op_call(attn_f, hid, Wo):
    n = D // _DB
    return pl.pallas_call(
        _op_body,
        grid=(n,),
        in_specs=[
            pl.BlockSpec((Q, D), lambda i: (0, 0)),
            pl.BlockSpec((Q, _DB), lambda i: (0, i)),
            pl.BlockSpec((_DB, D), lambda i: (i, 0)),
        ],
        out_specs=pl.BlockSpec((Q, _DB), lambda i: (0, i)),
        out_shape=jax.ShapeDtypeStruct((Q, D), F32),
    )(attn_f, hid, Wo)


def _mlp_body(h_ref, w2_ref, wg_ref, wu_ref, wd_ref, o_ref):
    i = pl.program_id(0)
    h = h_ref[...]
    hn = _rms(h, w2_ref[...])
    g = jax.nn.silu(_mm(hn, wg_ref[...], 1, 1))
    u = _mm(hn, wu_ref[...], 1, 1)
    part = _mm(g * u, wd_ref[...], 1, 1)

    @pl.when(i == 0)
    def _():
        o_ref[...] = h + part

    @pl.when(i > 0)
    def _():
        o_ref[...] += part


def _mlp_call(h_res, w2, Wg, Wu, Wd):
    n = FF // _FB
    return pl.pallas_call(
        _mlp_body,
        grid=(n,),
        in_specs=[
            pl.BlockSpec((Q, D), lambda i: (0, 0)),
            pl.BlockSpec((1, D), lambda i: (0, 0)),
            pl.BlockSpec((_FB, D), lambda i: (i, 0)),
            pl.BlockSpec((_FB, D), lambda i: (i, 0)),
            pl.BlockSpec((D, _FB), lambda i: (0, i)),
        ],
        out_specs=pl.BlockSpec((Q, D), lambda i: (0, 0)),
        out_shape=jax.ShapeDtypeStruct((Q, D), F32),
    )(h_res, w2, Wg, Wu, Wd)


def kernel(hidden_states, key_cache, value_cache, Wq, Wk, Wv, Wo,
           rot_mat1, rot_mat2, ln1_w, ln2_w, Wg, Wu, Wd):
    hid = hidden_states.reshape(Q, D)
    kc = key_cache.reshape(H, KV, HD)
    vc = value_cache.reshape(H, KV, HD)
    r1 = rot_mat1.reshape(H, HD, HD)
    r2 = rot_mat2.reshape(H, HD, HD)
    w1 = ln1_w.reshape(1, D)
    w2 = ln2_w.reshape(1, D)

    inv_freq = 1.0 / (ROPE_BASE ** (jnp.arange(0, HD, 2, dtype=F32) / HD))
    t = jnp.arange(L, dtype=F32)
    freqs = jnp.outer(t, inv_freq)
    emb = jnp.concatenate([freqs, freqs], axis=-1)
    cos = jnp.cos(emb)
    sin = jnp.sin(emb)
    cos_c, cos_n = cos[:KV], cos[KV:]
    sin_c, sin_n = sin[:KV], sin[KV:]

    q_f, k_f, v_f = _qkv_call(hid, w1, Wq, Wk, Wv)
    qh = q_f.reshape(Q, H, HD).transpose(1, 0, 2)
    kh = k_f.reshape(Q, H, HD).transpose(1, 0, 2)
    vh = v_f.reshape(Q, H, HD).transpose(1, 0, 2)
    v_new_pad = jnp.pad(vh, ((0, 0), (0, HD - Q), (0, 0)))

    q_rope, q_hash, draft_new, real_new = _hp_call(qh, kh, r1, r2, cos_n, sin_n)

    halves = []
    nh = H // 2
    for half in range(2):
        off = half * nh
        d_c, r_c = _score_call(kc, r1, r2, cos_c, sin_c, q_rope, q_hash,
                               off, nh)
        m_c, m_n = _sc_sel_call(d_c, draft_new[off:off + nh])
        halves.append((m_c, m_n, r_c, off))
    attn = jnp.concatenate(
        [_att_call(m_c, m_n, r_c, real_new[off:off + nh], vc, v_new_pad,
                   off, nh)
         for (m_c, m_n, r_c, off) in halves], axis=0)
    attn_f = attn.transpose(1, 0, 2).reshape(Q, D)
    h_res = _op_call(attn_f, hid, Wo)
    out = _mlp_call(h_res, w2, Wg, Wu, Wd)
    return out.reshape(B, Q, D)


# speedup vs baseline: 1.0380x; 1.0036x over previous
import functools

import jax
import jax.numpy as jnp
import numpy as np
from jax import lax
from jax.experimental import pallas as pl
from jax.experimental.pallas import tpu as pltpu
from jax.experimental.pallas import tpu_sc as plsc

B = 1; Q = 8; KV = 4096; H = 32; HD = 128; D = 4096; FF = 11008
L = KV + Q
LP = 4224
NBLK = LP // HD
NUM_REMAIN = L - int(L * 0.9)
ROPE_BASE = 10000.0
INV_SQRT_HD = 1.0 / np.sqrt(HD).astype(np.float32)
NEG = float(jnp.finfo(jnp.float32).min)
F32 = jnp.float32

_DB = 256
_FB = 256


def _rot_half(x):
    rolled = jnp.roll(x, HD // 2, axis=-1)
    lane = jax.lax.broadcasted_iota(jnp.int32, x.shape, len(x.shape) - 1)
    return jnp.where(lane < HD // 2, -rolled, rolled)


def _mm(a, b, ca, cb):
    return jax.lax.dot_general(a, b, (((ca,), (cb,)), ((), ())),
                               preferred_element_type=F32)


def _rms(x, w):
    ms = jnp.mean(x * x, axis=-1, keepdims=True)
    return x * jax.lax.rsqrt(ms + 1e-6) * w


def _qkv_body(h_ref, w1_ref, wq_ref, wk_ref, wv_ref, q_ref, k_ref, v_ref):
    hn = _rms(h_ref[...], w1_ref[...])
    q_ref[...] = _mm(hn, wq_ref[...], 1, 1)
    k_ref[...] = _mm(hn, wk_ref[...], 1, 1)
    v_ref[...] = _mm(hn, wv_ref[...], 1, 1)


def _qkv_call(hid, w1, Wq, Wk, Wv):
    n = D // _DB
    return pl.pallas_call(
        _qkv_body,
        grid=(n,),
        in_specs=[
            pl.BlockSpec((Q, D), lambda i: (0, 0)),
            pl.BlockSpec((1, D), lambda i: (0, 0)),
            pl.BlockSpec((_DB, D), lambda i: (i, 0)),
            pl.BlockSpec((_DB, D), lambda i: (i, 0)),
            pl.BlockSpec((_DB, D), lambda i: (i, 0)),
        ],
        out_specs=[pl.BlockSpec((Q, _DB), lambda i: (0, i))] * 3,
        out_shape=[jax.ShapeDtypeStruct((Q, D), F32)] * 3,
    )(hid, w1, Wq, Wk, Wv)


def _hp_body(q_ref, k_ref, r1_ref, r2_ref, cos_ref, sin_ref,
             qr_ref, qh_ref, dn_ref, rn_ref):
    q = q_ref[...].reshape(Q, HD)
    k = k_ref[...].reshape(Q, HD)
    cos = cos_ref[...]
    sin = sin_ref[...]
    r1 = r1_ref[...].reshape(HD, HD)
    r2 = r2_ref[...].reshape(HD, HD)
    qr = q * cos + _rot_half(q) * sin
    kr = k * cos + _rot_half(k) * sin
    qi = _mm(jax.nn.silu(_mm(qr, r1, 1, 0)), r2, 1, 0)
    ki = _mm(jax.nn.silu(_mm(kr, r1, 1, 0)), r2, 1, 0)
    qs = jnp.sign(qi)
    ks = jnp.sign(ki)
    qr_ref[...] = qr.reshape(1, Q, HD)
    qh_ref[...] = qs.reshape(1, Q, HD)
    dn_ref[...] = _mm(qs, ks, 1, 1).reshape(1, Q, Q)
    rn_ref[...] = (_mm(qr, kr, 1, 1) * INV_SQRT_HD).reshape(1, Q, Q)


def _hp_call(qh, kh, r1, r2, cos_n, sin_n):
    return pl.pallas_call(
        _hp_body,
        grid=(H,),
        in_specs=[
            pl.BlockSpec((1, Q, HD), lambda i: (i, 0, 0)),
            pl.BlockSpec((1, Q, HD), lambda i: (i, 0, 0)),
            pl.BlockSpec((1, HD, HD), lambda i: (i, 0, 0)),
            pl.BlockSpec((1, HD, HD), lambda i: (i, 0, 0)),
            pl.BlockSpec((Q, HD), lambda i: (0, 0)),
            pl.BlockSpec((Q, HD), lambda i: (0, 0)),
        ],
        out_specs=[
            pl.BlockSpec((1, Q, HD), lambda i: (i, 0, 0)),
            pl.BlockSpec((1, Q, HD), lambda i: (i, 0, 0)),
            pl.BlockSpec((1, Q, Q), lambda i: (i, 0, 0)),
            pl.BlockSpec((1, Q, Q), lambda i: (i, 0, 0)),
        ],
        out_shape=[
            jax.ShapeDtypeStruct((H, Q, HD), F32),
            jax.ShapeDtypeStruct((H, Q, HD), F32),
            jax.ShapeDtypeStruct((H, Q, Q), F32),
            jax.ShapeDtypeStruct((H, Q, Q), F32),
        ],
    )(qh, kh, r1, r2, cos_n, sin_n)


def _bd(a, b):
    z = jnp.zeros((HD, HD), F32)
    return jnp.concatenate(
        [jnp.concatenate([a, z], axis=1), jnp.concatenate([z, b], axis=1)],
        axis=0)


def _score_body(kc_ref, r1_ref, r2_ref, cos_ref, sin_ref, qr_ref, qh_ref,
                d_ref, r_ref):
    k2 = kc_ref[...]
    cos = cos_ref[...]
    sin = sin_ref[...]
    kra = k2[0] * cos + _rot_half(k2[0]) * sin
    krb = k2[1] * cos + _rot_half(k2[1]) * sin
    kr2 = jnp.concatenate([kra, krb], axis=1)
    r1 = r1_ref[...]
    r2 = r2_ref[...]
    bd1 = _bd(r1[0], r1[1])
    bd2 = _bd(r2[0], r2[1])
    ki2 = _mm(jax.nn.silu(_mm(kr2, bd1, 1, 0)), bd2, 1, 0)
    ks2 = jnp.sign(ki2)
    qh2 = qh_ref[...]
    qr2 = qr_ref[...]
    qhbd = _bd2q(qh2[0], qh2[1])
    qrbd = _bd2q(qr2[0], qr2[1])
    d_ref[...] = _mm(qhbd, ks2, 1, 1).reshape(2, Q, KV)
    r_ref[...] = (_mm(qrbd, kr2, 1, 1) * INV_SQRT_HD).reshape(2, Q, KV)


def _bd2q(a, b):
    z = jnp.zeros((Q, HD), F32)
    return jnp.concatenate(
        [jnp.concatenate([a, z], axis=1), jnp.concatenate([z, b], axis=1)],
        axis=0)


def _score_call(kc, r1, r2, cos_c, sin_c, q_rope, q_hash, off, nh):
    return pl.pallas_call(
        _score_body,
        grid=(nh // 2,),
        in_specs=[
            pl.BlockSpec((2, KV, HD), lambda i: (i + off // 2, 0, 0)),
            pl.BlockSpec((2, HD, HD), lambda i: (i + off // 2, 0, 0)),
            pl.BlockSpec((2, HD, HD), lambda i: (i + off // 2, 0, 0)),
            pl.BlockSpec((KV, HD), lambda i: (0, 0)),
            pl.BlockSpec((KV, HD), lambda i: (0, 0)),
            pl.BlockSpec((2, Q, HD), lambda i: (i + off // 2, 0, 0)),
            pl.BlockSpec((2, Q, HD), lambda i: (i + off // 2, 0, 0)),
        ],
        out_specs=[
            pl.BlockSpec((2, Q, KV), lambda i: (i, 0, 0)),
            pl.BlockSpec((2, Q, KV), lambda i: (i, 0, 0)),
        ],
        out_shape=[
            jax.ShapeDtypeStruct((nh, Q, KV), F32),
            jax.ShapeDtypeStruct((nh, Q, KV), F32),
        ],
    )(kc, r1, r2, cos_c, sin_c, q_rope, q_hash)


_ROWS = H * Q
_NW = 32
_RPW = _ROWS // _NW
_NBIN = 272
_NCH = KV // 16


def _sc_sel_body(rpw, dc_hbm, dn_hbm, mc_hbm, mn_hbm, row_v, tail_v, bins_v):
    wid = lax.axis_index("s") * 2 + lax.axis_index("c")
    lane = lax.iota(jnp.int32, 16)
    lanef = lane.astype(F32)
    lane_off = lane * _NBIN
    ones = jnp.full((16,), 1.0, F32)
    zeros = jnp.zeros((16,), F32)
    krem = float(NUM_REMAIN)
    nchunk = _NBIN // 16

    def _merged(i):
        m = bins_v[pl.ds(i * 16, 16)]
        for l in range(1, 16):
            m = m + bins_v[pl.ds(l * _NBIN + i * 16, 16)]
        return m

    def _row(r, row_carry):
        rg = wid * rpw + r
        tail_v[...] = jnp.full((16,), -1000.0, F32)
        pltpu.sync_copy(dc_hbm.at[pl.ds(rg * KV, KV)], row_v)
        pltpu.sync_copy(dn_hbm.at[pl.ds(rg * Q, Q)], tail_v.at[pl.ds(0, Q)])

        def _zero(i, c):
            bins_v[pl.ds(i * 16, 16)] = zeros
            return c

        lax.fori_loop(0, _NBIN, _zero, 0, unroll=4)

        def _hist(c, carry):
            v = row_v[pl.ds(c * 16, 16)]
            idx = v.astype(jnp.int32) + 128 + lane_off
            plsc.addupdate_scatter(bins_v, [idx], ones)
            return carry

        lax.fori_loop(0, _NCH, _hist, 0, unroll=8)
        tv = tail_v[...]
        tidx = tv.astype(jnp.int32) + 128 + lane_off
        plsc.addupdate_scatter(bins_v, [tidx], ones, mask=tv > -999.0)

        def _scan(j, st):
            t_run, cge_run, carry = st
            i = nchunk - 1 - j
            m = _merged(i)
            cs = plsc.cumsum(lax.rev(m, (0,))) + carry
            bv = (i * 16 + 15 - 128).astype(F32) - lanef
            ok = cs >= krem
            t_run = jnp.maximum(t_run, jnp.where(ok, bv, -999.0))
            cge_run = jnp.minimum(cge_run, jnp.where(ok, cs, 1e9))
            return t_run, cge_run, carry + jnp.sum(m)

        t_run, cge_run, _ = lax.fori_loop(
            0, nchunk, _scan,
            (jnp.full((16,), -999.0, F32), jnp.full((16,), 1e9, F32),
             jnp.float32(0.0)))
        thr = jnp.max(t_run)
        cge = jnp.min(cge_run)
        bidx = thr.astype(jnp.int32) + 128
        cbase = (bidx // 16) * 16
        blane = bidx - cbase

        def _ecnt(l, acc):
            m = bins_v[pl.ds(l * _NBIN + cbase, 16)]
            return acc + jnp.sum(jnp.where(lane == blane, m, 0.0))

        ecnt = lax.fori_loop(0, 16, _ecnt, jnp.float32(0.0))
        rrem = krem - (cge - ecnt)

        def _sel(c, base):
            v = row_v[pl.ds(c * 16, 16)]
            eq = jnp.where(v == thr, 1.0, 0.0)
            csum = plsc.cumsum(eq) + base
            keep = (v > thr) | ((eq > 0.5) & (csum <= rrem))
            row_v[pl.ds(c * 16, 16)] = jnp.where(keep, 0.0, NEG)
            return base + jnp.sum(eq)

        base = lax.fori_loop(0, _NCH, _sel, jnp.float32(0.0), unroll=8)
        tv = tail_v[...]
        eq = jnp.where(tv == thr, 1.0, 0.0)
        csum = plsc.cumsum(eq) + base
        keep = (tv > thr) | ((eq > 0.5) & (csum <= rrem))
        tail_v[...] = jnp.where(keep, 0.0, NEG)

        pltpu.sync_copy(row_v, mc_hbm.at[pl.ds(rg * KV, KV)])
        pltpu.sync_copy(tail_v.at[pl.ds(0, Q)], mn_hbm.at[pl.ds(rg * Q, Q)])
        return row_carry

    lax.fori_loop(0, rpw, _row, 0)


def _sc_sel_call(draft_c, draft_new):
    nh = draft_c.shape[0]
    rows = nh * Q
    rpw = rows // _NW
    fn = functools.partial(
        pl.kernel,
        out_type=[
            jax.ShapeDtypeStruct((rows * KV,), F32),
            jax.ShapeDtypeStruct((rows * Q,), F32),
        ],
        mesh=plsc.VectorSubcoreMesh(core_axis_name="c", subcore_axis_name="s"),
        compiler_params=pltpu.CompilerParams(needs_layout_passes=False),
        scratch_types=[
            pltpu.VMEM((KV,), F32),
            pltpu.VMEM((16,), F32),
            pltpu.VMEM((16 * _NBIN,), F32),
        ],
    )(functools.partial(_sc_sel_body, rpw))
    mc, mn = fn(draft_c.reshape(rows * KV), draft_new.reshape(rows * Q))
    return mc.reshape(nh, Q, KV), mn.reshape(nh, Q, Q)


_AH = 4
_AR = _AH * Q


def _att_body(mc_ref, mn_ref, rc_ref, rn_ref, vc_ref, vn_ref, o_ref):
    mc = mc_ref[...].reshape(_AR, KV)
    mn = mn_ref[...].reshape(_AR, Q)
    rc = rc_ref[...].reshape(_AR, KV)
    rn = rn_ref[...].reshape(_AR, Q)
    masked = jnp.concatenate(
        [rc + mc, rn + mn, jnp.full((_AR, LP - L), NEG, F32)], axis=-1)
    m = jnp.max(masked, axis=-1, keepdims=True)
    p = jnp.exp(masked - m)
    denom = jnp.sum(p, axis=-1, keepdims=True)
    vc = vc_ref[...]
    vn = vn_ref[...]
    outs = []
    for hh in range(_AH):
        ph = p[hh * Q:(hh + 1) * Q]
        att = _mm(ph[:, :KV], vc[hh], 1, 0) + _mm(ph[:, KV:LP], vn[hh], 1, 0)
        outs.append(att / denom[hh * Q:(hh + 1) * Q])
    o_ref[...] = jnp.concatenate(outs, axis=0).reshape(_AH, Q, HD)


def _att_call(mask_c, mask_new, real_c, real_new, vc, v_new_pad, off, nh):
    return pl.pallas_call(
        _att_body,
        grid=(nh // _AH,),
        in_specs=[
            pl.BlockSpec((_AH, Q, KV), lambda i: (i, 0, 0)),
            pl.BlockSpec((_AH, Q, Q), lambda i: (i, 0, 0)),
            pl.BlockSpec((_AH, Q, KV), lambda i: (i, 0, 0)),
            pl.BlockSpec((_AH, Q, Q), lambda i: (i, 0, 0)),
            pl.BlockSpec((_AH, KV, HD), lambda i: (i + off // _AH, 0, 0)),
            pl.BlockSpec((_AH, HD, HD), lambda i: (i + off // _AH, 0, 0)),
        ],
        out_specs=pl.BlockSpec((_AH, Q, HD), lambda i: (i, 0, 0)),
        out_shape=jax.ShapeDtypeStruct((nh, Q, HD), F32),
    )(mask_c, mask_new, real_c, real_new, vc, v_new_pad)


_NOP = D // _DB
_NFF = FF // _FB


def _tail_body(a_ref, hid_ref, w2_ref, wo_ref, wg_ref, wu_ref, wd_ref,
               o_ref, hres_s):
    i = pl.program_id(0)

    @pl.when(i < _NOP)
    def _():
        blk = _mm(a_ref[...], wo_ref[...], 1, 1) + hid_ref[...]
        off = pl.multiple_of(i * _DB, _DB)
        hres_s[:, pl.ds(off, _DB)] = blk

    @pl.when(i >= _NOP)
    def _():
        h = hres_s[...]
        hn = _rms(h, w2_ref[...])
        g = jax.nn.silu(_mm(hn, wg_ref[...], 1, 1))
        u = _mm(hn, wu_ref[...], 1, 1)
        part = _mm(g * u, wd_ref[...], 1, 1)

        @pl.when(i == _NOP)
        def _():
            o_ref[...] = h + part

        @pl.when(i > _NOP)
        def _():
            o_ref[...] += part


def _tail_call(attn_f, hid, w2, Wo, Wg, Wu, Wd):
    return pl.pallas_call(
        _tail_body,
        grid=(_NOP + _NFF,),
        in_specs=[
            pl.BlockSpec((Q, D), lambda i: (0, 0)),
            pl.BlockSpec((Q, _DB), lambda i: (0, jnp.minimum(i, _NOP - 1))),
            pl.BlockSpec((1, D), lambda i: (0, 0)),
            pl.BlockSpec((_DB, D), lambda i: (jnp.minimum(i, _NOP - 1), 0)),
            pl.BlockSpec((_FB, D), lambda i: (jnp.maximum(i - _NOP, 0), 0)),
            pl.BlockSpec((_FB, D), lambda i: (jnp.maximum(i - _NOP, 0), 0)),
            pl.BlockSpec((D, _FB), lambda i: (0, jnp.maximum(i - _NOP, 0))),
        ],
        out_specs=pl.BlockSpec((Q, D), lambda i: (0, 0)),
        out_shape=jax.ShapeDtypeStruct((Q, D), F32),
        scratch_shapes=[pltpu.VMEM((Q, D), F32)],
    )(attn_f, hid, w2, Wo, Wg, Wu, Wd)


def kernel(hidden_states, key_cache, value_cache, Wq, Wk, Wv, Wo,
           rot_mat1, rot_mat2, ln1_w, ln2_w, Wg, Wu, Wd):
    hid = hidden_states.reshape(Q, D)
    kc = key_cache.reshape(H, KV, HD)
    vc = value_cache.reshape(H, KV, HD)
    r1 = rot_mat1.reshape(H, HD, HD)
    r2 = rot_mat2.reshape(H, HD, HD)
    w1 = ln1_w.reshape(1, D)
    w2 = ln2_w.reshape(1, D)

    inv_freq = 1.0 / (ROPE_BASE ** (jnp.arange(0, HD, 2, dtype=F32) / HD))
    t = jnp.arange(L, dtype=F32)
    freqs = jnp.outer(t, inv_freq)
    emb = jnp.concatenate([freqs, freqs], axis=-1)
    cos = jnp.cos(emb)
    sin = jnp.sin(emb)
    cos_c, cos_n = cos[:KV], cos[KV:]
    sin_c, sin_n = sin[:KV], sin[KV:]

    q_f, k_f, v_f = _qkv_call(hid, w1, Wq, Wk, Wv)
    qh = q_f.reshape(Q, H, HD).transpose(1, 0, 2)
    kh = k_f.reshape(Q, H, HD).transpose(1, 0, 2)
    vh = v_f.reshape(Q, H, HD).transpose(1, 0, 2)
    v_new_pad = jnp.pad(vh, ((0, 0), (0, HD - Q), (0, 0)))

    q_rope, q_hash, draft_new, real_new = _hp_call(qh, kh, r1, r2, cos_n, sin_n)

    halves = []
    nh = H // 2
    for half in range(2):
        off = half * nh
        d_c, r_c = _score_call(kc, r1, r2, cos_c, sin_c, q_rope, q_hash,
                               off, nh)
        m_c, m_n = _sc_sel_call(d_c, draft_new[off:off + nh])
        halves.append((m_c, m_n, r_c, off))
    attn = jnp.concatenate(
        [_att_call(m_c, m_n, r_c, real_new[off:off + nh], vc, v_new_pad,
                   off, nh)
         for (m_c, m_n, r_c, off) in halves], axis=0)
    attn_f = attn.transpose(1, 0, 2).reshape(Q, D)
    out = _tail_call(attn_f, hid, w2, Wo, Wg, Wu, Wd)
    return out.reshape(B, Q, D)
